# Initial kernel scaffold; baseline (speedup 1.0000x reference)
import jax
import jax.numpy as jnp
from jax.experimental import pallas as pl


def kernel(indices, per_atom_property):
    raise NotImplementedError("write your pallas kernel here")


# speedup vs baseline: 28.5836x; 28.5836x over previous
import jax
import jax.numpy as jnp
from jax import lax
from jax.experimental import pallas as pl
from jax.experimental.pallas import tpu as pltpu
from jax.experimental.pallas import tpu_sc as plsc

NA = 6_400_000
NM = 100_000
NMP = 100_352
NC = 2
NS = 16
APW = NA // (NC * NS)
PIECE = 8_000
NPIECE = APW // PIECE
ZCH = NMP // NS


def _partials_kernel(idx_hbm, val_hbm, part_hbm, idxbuf, valbuf, acc):
    c = lax.axis_index("c")
    s = lax.axis_index("s")
    wid = c * NS + s

    def zero_body(j, _):
        valbuf[pl.ds(16 * j, 16)] = jnp.zeros((16,), jnp.float32)
        return _

    lax.fori_loop(0, ZCH // 16, zero_body, None)
    pltpu.sync_copy(valbuf.at[pl.ds(0, ZCH)], acc.at[pl.ds(s * ZCH, ZCH)])
    plsc.subcore_barrier()

    def piece_body(i, _):
        base = wid * APW + i * PIECE
        pltpu.sync_copy(idx_hbm.at[pl.ds(base, PIECE)], idxbuf)
        pltpu.sync_copy(val_hbm.at[pl.ds(base, PIECE)], valbuf)
        pltpu.sync_copy(valbuf, acc.at[idxbuf], add=True)
        return _

    lax.fori_loop(0, NPIECE, piece_body, None)
    plsc.subcore_barrier()

    pltpu.sync_copy(acc.at[pl.ds(s * ZCH, ZCH)],
                    part_hbm.at[pl.ds(c * NMP + s * ZCH, ZCH)])


def _combine_kernel(part_hbm, out_hbm, bufa, bufb):
    c = lax.axis_index("c")
    s = lax.axis_index("s")
    w = c * NS + s
    ch = NMP // (NC * NS)
    base = w * ch
    pltpu.sync_copy(part_hbm.at[pl.ds(base, ch)], bufa)
    pltpu.sync_copy(part_hbm.at[pl.ds(NMP + base, ch)], bufb)

    def add_body(j, _):
        sl = pl.ds(16 * j, 16)
        bufa[sl] = bufa[sl] + bufb[sl]
        return _

    lax.fori_loop(0, ch // 16, add_body, None)
    pltpu.sync_copy(bufa, out_hbm.at[pl.ds(base, ch)])


def kernel(indices, per_atom_property):
    mesh = plsc.VectorSubcoreMesh(core_axis_name="c", subcore_axis_name="s")

    partials = pl.kernel(
        _partials_kernel,
        out_type=jax.ShapeDtypeStruct((NC * NMP,), jnp.float32),
        mesh=mesh,
        scratch_types=[
            pltpu.VMEM((PIECE,), jnp.int32),
            pltpu.VMEM((PIECE,), jnp.float32),
            pltpu.VMEM_SHARED((NMP,), jnp.float32),
        ],
    )(indices, per_atom_property)

    out = pl.kernel(
        _combine_kernel,
        out_type=jax.ShapeDtypeStruct((NMP,), jnp.float32),
        mesh=mesh,
        scratch_types=[
            pltpu.VMEM((NMP // (NC * NS),), jnp.float32),
            pltpu.VMEM((NMP // (NC * NS),), jnp.float32),
        ],
    )(partials)

    return out[:NM]
